# baseline (speedup 1.0000x reference)
import jax
import jax.numpy as jnp
from jax.experimental import pallas as pl

B = 2
C = 20
LAMBDA_COORD = 5.0
LAMBDA_NOOBJ = 0.5

_NCELLS = 16384 * 49
_BLK = 1568


def _iou_pair(px, py, pw2, ph2, tx, ty, tw2, th2):
    p1x, p1y = px - pw2, py - ph2
    p2x, p2y = px + pw2, py + ph2
    t1x, t1y = tx - tw2, ty - th2
    t2x, t2y = tx + tw2, ty + th2
    tlx = jnp.maximum(p1x, t1x)
    tly = jnp.maximum(p1y, t1y)
    brx = jnp.minimum(p2x, t2x)
    bry = jnp.minimum(p2y, t2y)
    wx = jnp.maximum(brx - tlx, 0.0)
    wy = jnp.maximum(bry - tly, 0.0)
    inter = wx * wy
    area1 = (p2x - p1x) * (p2y - p1y)
    area2 = (t2x - t1x) * (t2y - t1y)
    return inter / (area1 + area2 - inter)


def _block_body(p_ref, t_ref, o_ref):
    p = p_ref[...]
    t = t_ref[...]
    cw = (t[:, 5] > 0).astype(jnp.float32)
    nw = (t[:, 5] == 0).astype(jnp.float32)
    d = p - t
    d2 = d * d
    class_sum = jnp.sum(d2[:, 10:], axis=1)
    noobj_sum = d2[:, 4] + d2[:, 9]
    loc0 = d2[:, 0] + d2[:, 1] + d2[:, 2] + d2[:, 3]
    loc1 = d2[:, 5] + d2[:, 6] + d2[:, 7] + d2[:, 8]

    px = (p[:, 0], p[:, 5])
    py = (p[:, 1], p[:, 6])
    pw2 = (p[:, 2] * p[:, 2], p[:, 7] * p[:, 7])
    ph2 = (p[:, 3] * p[:, 3], p[:, 8] * p[:, 8])
    tx = (t[:, 0], t[:, 5])
    ty = (t[:, 1], t[:, 6])
    tw2 = (t[:, 2] * t[:, 2], t[:, 7] * t[:, 7])
    th2 = (t[:, 3] * t[:, 3], t[:, 8] * t[:, 8])

    iou = [[_iou_pair(px[i], py[i], pw2[i], ph2[i],
                      tx[j], ty[j], tw2[j], th2[j]) for j in range(2)]
           for i in range(2)]
    m0 = iou[1][0] > iou[0][0]
    m1 = iou[1][1] > iou[0][1]
    resp0 = jnp.logical_or(jnp.logical_not(m0), jnp.logical_not(m1))
    resp1 = jnp.logical_or(m0, m1)
    w0 = cw * resp0.astype(jnp.float32)
    w1 = cw * resp1.astype(jnp.float32)

    rows = (cw * class_sum
            + LAMBDA_NOOBJ * nw * noobj_sum
            + w0 * d2[:, 4] + w1 * d2[:, 9]
            + LAMBDA_COORD * (w0 * loc0 + w1 * loc1))
    part = jnp.sum(rows).reshape(1, 1)

    @pl.when(pl.program_id(0) == 0)
    def _init():
        o_ref[...] = jnp.zeros((1, 1), jnp.float32)

    o_ref[...] += part


def kernel(pred_tensor, target_tensor):
    p = pred_tensor.reshape(_NCELLS, 30)
    t = target_tensor.reshape(_NCELLS, 30)
    grid = _NCELLS // _BLK
    out = pl.pallas_call(
        _block_body,
        grid=(grid,),
        in_specs=[
            pl.BlockSpec((_BLK, 30), lambda i: (i, 0)),
            pl.BlockSpec((_BLK, 30), lambda i: (i, 0)),
        ],
        out_specs=pl.BlockSpec((1, 1), lambda i: (0, 0)),
        out_shape=jax.ShapeDtypeStruct((1, 1), jnp.float32),
    )(p, t)
    return out[0, 0]


# speedup vs baseline: 3.3338x; 3.3338x over previous
import jax
import jax.numpy as jnp
import numpy as np
from jax import lax
from jax.experimental import pallas as pl

LAMBDA_COORD = 5.0
LAMBDA_NOOBJ = 0.5

_NCELLS = 16384 * 49
_BLK = 4096

def _seg_selector():
    e = lax.broadcasted_iota(jnp.int32, (30, 5), 0)
    k = lax.broadcasted_iota(jnp.int32, (30, 5), 1)
    sel = ((k == 0) & (e < 4)) | ((k == 1) & (e >= 5) & (e < 9)) | \
          ((k == 2) & (e == 4)) | ((k == 3) & (e == 9)) | \
          ((k == 4) & (e >= 10))
    return sel.astype(jnp.float32)


def _corners(a, i):
    x = a[5 * i:5 * i + 1, :]
    y = a[5 * i + 1:5 * i + 2, :]
    w = a[5 * i + 2:5 * i + 3, :]
    h = a[5 * i + 3:5 * i + 4, :]
    w2 = w * w
    h2 = h * h
    return x - w2, y - h2, x + w2, y + h2


def _iou(b1, b2):
    tlx = jnp.maximum(b1[0], b2[0])
    tly = jnp.maximum(b1[1], b2[1])
    brx = jnp.minimum(b1[2], b2[2])
    bry = jnp.minimum(b1[3], b2[3])
    wx = jnp.maximum(brx - tlx, 0.0)
    wy = jnp.maximum(bry - tly, 0.0)
    inter = wx * wy
    a1 = (b1[2] - b1[0]) * (b1[3] - b1[1])
    a2 = (b2[2] - b2[0]) * (b2[3] - b2[1])
    return inter / (a1 + a2 - inter)


def _block_body(p_ref, t_ref, o_ref):
    p = p_ref[...]
    t = t_ref[...]
    d = p - t
    d2 = d * d
    sel = _seg_selector()
    s = jnp.dot(d2, sel, preferred_element_type=jnp.float32)

    pt = jnp.transpose(p[:, 0:10])
    tt = jnp.transpose(t[:, 0:10])

    pb = (_corners(pt, 0), _corners(pt, 1))
    tb = (_corners(tt, 0), _corners(tt, 1))
    iou = [[_iou(pb[i], tb[j]) for j in range(2)] for i in range(2)]
    m0 = iou[1][0] > iou[0][0]
    m1 = iou[1][1] > iou[0][1]
    resp0 = jnp.logical_or(jnp.logical_not(m0), jnp.logical_not(m1))
    resp1 = jnp.logical_or(m0, m1)

    t5 = tt[5:6, :]
    cw = (t5 > 0).astype(jnp.float32)
    nw = (t5 == 0).astype(jnp.float32)
    w0 = cw * resp0.astype(jnp.float32)
    w1 = cw * resp1.astype(jnp.float32)

    coeff = jnp.concatenate(
        [LAMBDA_COORD * w0,
         LAMBDA_COORD * w1,
         w0 + LAMBDA_NOOBJ * nw,
         w1 + LAMBDA_NOOBJ * nw,
         cw], axis=0)

    cs = jnp.dot(coeff, s, preferred_element_type=jnp.float32)
    r = lax.broadcasted_iota(jnp.int32, (5, 5), 0)
    c = lax.broadcasted_iota(jnp.int32, (5, 5), 1)
    part = jnp.sum(jnp.where(r == c, cs, 0.0)).reshape(1, 1)

    @pl.when(pl.program_id(0) == 0)
    def _init():
        o_ref[...] = jnp.zeros((1, 1), jnp.float32)

    o_ref[...] += part


def kernel(pred_tensor, target_tensor):
    p = pred_tensor.reshape(_NCELLS, 30)
    t = target_tensor.reshape(_NCELLS, 30)
    grid = _NCELLS // _BLK
    out = pl.pallas_call(
        _block_body,
        grid=(grid,),
        in_specs=[
            pl.BlockSpec((_BLK, 30), lambda i: (i, 0)),
            pl.BlockSpec((_BLK, 30), lambda i: (i, 0)),
        ],
        out_specs=pl.BlockSpec((1, 1), lambda i: (0, 0)),
        out_shape=jax.ShapeDtypeStruct((1, 1), jnp.float32),
    )(p, t)
    return out[0, 0]


# speedup vs baseline: 10.9244x; 3.2769x over previous
import jax
import jax.numpy as jnp
import numpy as np
from jax import lax
from jax.experimental import pallas as pl

LAMBDA_COORD = 5.0
LAMBDA_NOOBJ = 0.5

_ROWS = 16384
_COLS = 1470
_R = 512
_G = 64

_BOX_ELEMS = (0, 1, 2, 3, 5, 6, 7, 8)
_EBOX = np.zeros((_COLS, 8 * _G), np.float32)
for g, e in enumerate(_BOX_ELEMS):
    for c in range(49):
        _EBOX[30 * c + e, g * _G + c] = 1.0

_SEGS = ((0, 1, 2, 3), (5, 6, 7, 8), (4,), (9,), tuple(range(10, 30)))
_ESEG = np.zeros((_COLS, 5 * _G), np.float32)
for k, seg in enumerate(_SEGS):
    for c in range(49):
        for e in seg:
            _ESEG[30 * c + e, k * _G + c] = 1.0


def _iou(b1, b2):
    tlx = jnp.maximum(b1[0], b2[0])
    tly = jnp.maximum(b1[1], b2[1])
    brx = jnp.minimum(b1[2], b2[2])
    bry = jnp.minimum(b1[3], b2[3])
    wx = jnp.maximum(brx - tlx, 0.0)
    wy = jnp.maximum(bry - tly, 0.0)
    inter = wx * wy
    a1 = (b1[2] - b1[0]) * (b1[3] - b1[1])
    a2 = (b2[2] - b2[0]) * (b2[3] - b2[1])
    return inter / (a1 + a2 - inter)


def _boxes(xb):
    def grp(i):
        return xb[:, i * _G:(i + 1) * _G]
    out = []
    for i in range(2):
        x, y, w, h = grp(4 * i), grp(4 * i + 1), grp(4 * i + 2), grp(4 * i + 3)
        w2 = w * w
        h2 = h * h
        out.append((x - w2, y - h2, x + w2, y + h2))
    return out


def _block_body(p_ref, t_ref, ebox_ref, eseg_ref, o_ref):
    p = p_ref[...]
    t = t_ref[...]
    d = p - t
    d2b = (d * d).astype(jnp.bfloat16)
    pb = jnp.dot(p.astype(jnp.bfloat16), ebox_ref[...],
                 preferred_element_type=jnp.float32)
    tb = jnp.dot(t.astype(jnp.bfloat16), ebox_ref[...],
                 preferred_element_type=jnp.float32)
    segs = jnp.dot(d2b, eseg_ref[...],
                   preferred_element_type=jnp.float32)

    pboxes = _boxes(pb)
    tboxes = _boxes(tb)
    iou = [[_iou(pboxes[i], tboxes[j]) for j in range(2)] for i in range(2)]
    m0 = iou[1][0] > iou[0][0]
    m1 = iou[1][1] > iou[0][1]
    resp0 = jnp.logical_or(jnp.logical_not(m0), jnp.logical_not(m1))
    resp1 = jnp.logical_or(m0, m1)

    t5 = tb[:, 4 * _G:5 * _G]
    cw = (t5 > 0).astype(jnp.float32)
    nw = (t5 == 0).astype(jnp.float32)
    w0 = cw * resp0.astype(jnp.float32)
    w1 = cw * resp1.astype(jnp.float32)

    coeff = jnp.concatenate(
        [LAMBDA_COORD * w0,
         LAMBDA_COORD * w1,
         w0 + LAMBDA_NOOBJ * nw,
         w1 + LAMBDA_NOOBJ * nw,
         cw], axis=1)

    part = jnp.sum(coeff * segs).reshape(1, 1)

    @pl.when(pl.program_id(0) == 0)
    def _init():
        o_ref[...] = jnp.zeros((1, 1), jnp.float32)

    o_ref[...] += part


def kernel(pred_tensor, target_tensor):
    t2 = target_tensor.reshape(_ROWS, _COLS)
    ebox = jnp.asarray(_EBOX, jnp.bfloat16)
    eseg = jnp.asarray(_ESEG, jnp.bfloat16)
    grid = _ROWS // _R
    out = pl.pallas_call(
        _block_body,
        grid=(grid,),
        in_specs=[
            pl.BlockSpec((_R, _COLS), lambda i: (i, 0)),
            pl.BlockSpec((_R, _COLS), lambda i: (i, 0)),
            pl.BlockSpec((_COLS, 8 * _G), lambda i: (0, 0)),
            pl.BlockSpec((_COLS, 5 * _G), lambda i: (0, 0)),
        ],
        out_specs=pl.BlockSpec((1, 1), lambda i: (0, 0)),
        out_shape=jax.ShapeDtypeStruct((1, 1), jnp.float32),
    )(pred_tensor, t2, ebox, eseg)
    return out[0, 0]
